# baseline (speedup 1.0000x reference)
import dataclasses

import jax
import jax.numpy as jnp
from jax import lax
from jax.experimental import pallas as pl
from jax.experimental.pallas import tpu as pltpu
from jax.experimental.pallas import tpu_sc as plsc

N = 10000
NPAD = 10240
F = 128
H = 256
O = 64
E = 160000
EPAD = 163840
NC, NS, L = 2, 16, 16
CHUNK = 128
ET = EPAD // (NC * NS)
NCH = ET // CHUNK
RPT = NPAD // NS

_mesh = plsc.VectorSubcoreMesh(core_axis_name="c", subcore_axis_name="s")
_SC_CP = pltpu.CompilerParams()
if "needs_layout_passes" in pltpu.CompilerParams.__dataclass_fields__:
    _SC_CP = dataclasses.replace(_SC_CP, needs_layout_passes=False)
f32 = jnp.float32
i32 = jnp.int32


def _zero_rows_f(tb_v):
    k = tb_v.shape[1] // L
    zf = jnp.zeros((L,), f32)

    @pl.loop(0, 128)
    def _(r):
        for fblk in range(k):
            tb_v[r, pl.ds(fblk * L, L)] = zf


def _k0_body(dstrs_hbm, dstrr_hbm, w_hbm,
             deg_out, cnt_out,
             dst_v, dstr_v, w_v, ones_v, st1_v,
             acc_deg, acc_cnt):
    c = lax.axis_index("c")
    s = lax.axis_index("s")
    wid = c * NS + s
    zf = jnp.zeros((L,), f32)
    of = jnp.ones((L,), f32)
    base = s * RPT

    @pl.loop(0, RPT, step=L)
    def _(r):
        st1_v[pl.ds(r, L)] = zf

    @pl.loop(0, CHUNK, step=L)
    def _(r):
        ones_v[pl.ds(r, L)] = of

    pltpu.sync_copy(st1_v, acc_deg.at[pl.ds(base, RPT)])
    pltpu.sync_copy(st1_v, acc_cnt.at[pl.ds(base, RPT)])

    pltpu.sync_copy(dstrs_hbm.at[wid], dst_v)
    pltpu.sync_copy(dstrr_hbm.at[wid], dstr_v)
    pltpu.sync_copy(w_hbm.at[pl.ds(wid * ET, ET)], w_v)

    plsc.subcore_barrier()

    @pl.loop(0, NCH)
    def _(j):
        pltpu.sync_copy(w_v.at[pl.ds(j * CHUNK, CHUNK)],
                        acc_deg.at[dstr_v.at[j]], add=True)
        pltpu.sync_copy(ones_v, acc_cnt.at[dst_v.at[j]], add=True)

    plsc.subcore_barrier()

    pltpu.sync_copy(acc_deg.at[pl.ds(base, RPT)], st1_v)
    pltpu.sync_copy(st1_v, deg_out.at[c, pl.ds(base, RPT)])
    pltpu.sync_copy(acc_cnt.at[pl.ds(base, RPT)], st1_v)
    pltpu.sync_copy(st1_v, cnt_out.at[c, pl.ds(base, RPT)])


@jax.jit
def _k0(dst_rs2d, dst_rr2d, w_pad):
    kern = pl.kernel(
        _k0_body,
        mesh=_mesh,
        compiler_params=_SC_CP,
        out_type=(
            jax.ShapeDtypeStruct((NC, NPAD), f32),
            jax.ShapeDtypeStruct((NC, NPAD), f32),
        ),
        scratch_types=[
            pltpu.VMEM((NCH, CHUNK), i32),
            pltpu.VMEM((NCH, CHUNK), i32),
            pltpu.VMEM((ET,), f32),
            pltpu.VMEM((CHUNK,), f32),
            pltpu.VMEM((RPT,), f32),
            pltpu.VMEM_SHARED((NPAD,), f32),
            pltpu.VMEM_SHARED((NPAD,), f32),
        ],
    )
    return kern(dst_rs2d, dst_rr2d, w_pad)


def _k1_body(x_hbm, srcrs_hbm, dstrs_hbm, tok_hbm,
             s0_out,
             src_v, dst_v, r0,
             acc_s0, ss):
    c = lax.axis_index("c")
    s = lax.axis_index("s")
    wid = c * NS + s
    base = s * RPT

    _zero_rows_f(r0)
    for r in range(RPT // 128):
        pltpu.sync_copy(r0, acc_s0.at[pl.ds(base + r * 128, 128)])

    pltpu.sync_copy(srcrs_hbm.at[pl.ds(wid * ET, ET)], src_v)
    pltpu.sync_copy(dstrs_hbm.at[wid], dst_v)

    plsc.subcore_barrier()

    @pl.loop(0, NCH)
    def _(j):
        pltpu.async_copy(x_hbm.at[src_v.at[pl.ds(j * CHUNK, CHUNK)]], r0, ss).wait()
        pltpu.sync_copy(r0, acc_s0.at[dst_v.at[j]], add=True)

    plsc.subcore_barrier()

    for r in range(RPT // 128):
        row0 = base + r * 128
        pltpu.sync_copy(acc_s0.at[pl.ds(row0, 128)], r0)
        pltpu.sync_copy(r0, s0_out.at[c, pl.ds(row0, 128)])


@jax.jit
def _k1(x_pad, src_rs, dst_rs2d, tok):
    kern = pl.kernel(
        _k1_body,
        mesh=_mesh,
        compiler_params=_SC_CP,
        out_type=jax.ShapeDtypeStruct((NC, NPAD, F), f32),
        scratch_types=[
            pltpu.VMEM((ET,), i32),
            pltpu.VMEM((NCH, CHUNK), i32),
            pltpu.VMEM((CHUNK, F), f32),
            pltpu.VMEM_SHARED((NPAD, F), f32),
            pltpu.SemaphoreType.DMA,
        ],
    )
    return kern(x_pad, src_rs, dst_rs2d, tok)


def _k1b_body(src_hbm, w_hbm, dis_hbm, tok_hbm,
              s_out,
              src_v, w_v, dis_v, st_v):
    c = lax.axis_index("c")
    s = lax.axis_index("s")
    wid = c * NS + s

    pltpu.sync_copy(src_hbm.at[pl.ds(wid * ET, ET)], src_v)
    pltpu.sync_copy(w_hbm.at[pl.ds(wid * ET, ET)], w_v)
    pltpu.sync_copy(dis_hbm, dis_v)

    @pl.loop(0, ET, step=L)
    def _(i):
        idx16 = src_v[pl.ds(i, L)]
        d16 = plsc.load_gather(dis_v, [idx16])
        st_v[pl.ds(i, L)] = d16 * w_v[pl.ds(i, L)]

    pltpu.sync_copy(st_v, s_out.at[pl.ds(wid * ET, ET)])


@jax.jit
def _k1b(src_rr, w_pad, dis_flat, tok):
    kern = pl.kernel(
        _k1b_body,
        mesh=_mesh,
        compiler_params=_SC_CP,
        out_type=jax.ShapeDtypeStruct((EPAD,), f32),
        scratch_types=[
            pltpu.VMEM((ET,), i32),
            pltpu.VMEM((ET,), f32),
            pltpu.VMEM((NPAD,), f32),
            pltpu.VMEM((ET,), f32),
        ],
    )
    return kern(src_rr, w_pad, dis_flat, tok)


EPH = EPAD // 2
ETH = EPH // (NC * NS)
NCHH = ETH // CHUNK


def _k2_body(x_hbm, src_hbm, dst_hbm, s_hbm,
             r0_out,
             src_v, dst_v, sch_v, r0,
             acc, ss):
    c = lax.axis_index("c")
    s = lax.axis_index("s")
    wid = c * NS + s
    zi = jnp.zeros((L,), i32)
    base = s * RPT

    _zero_rows_f(r0)
    for r in range(RPT // 128):
        pltpu.sync_copy(r0, acc.at[pl.ds(base + r * 128, 128)])

    pltpu.sync_copy(src_hbm.at[pl.ds(wid * ETH, ETH)], src_v)
    pltpu.sync_copy(dst_hbm.at[wid], dst_v)
    pltpu.sync_copy(s_hbm.at[pl.ds(wid * ETH, ETH)], sch_v)

    plsc.subcore_barrier()

    @pl.loop(0, NCHH)
    def _(j):
        pltpu.async_copy(x_hbm.at[src_v.at[pl.ds(j * CHUNK, CHUNK)]], r0, ss).wait()

        @pl.loop(0, CHUNK)
        def _(e):
            sp = plsc.load_gather(sch_v, [zi + (j * CHUNK + e)])
            for fblk in range(F // L):
                r0[e, pl.ds(fblk * L, L)] = r0[e, pl.ds(fblk * L, L)] * sp

        pltpu.sync_copy(r0, acc.at[dst_v.at[j]], add=True)

    plsc.subcore_barrier()

    for r in range(RPT // 128):
        row0 = base + r * 128
        pltpu.sync_copy(acc.at[pl.ds(row0, 128)], r0)
        pltpu.sync_copy(r0, r0_out.at[c, pl.ds(row0, 128)])


@jax.jit
def _k2(x_pad, src_h, dst2d_h, s_h):
    kern = pl.kernel(
        _k2_body,
        mesh=_mesh,
        compiler_params=_SC_CP,
        out_type=jax.ShapeDtypeStruct((NC, NPAD, F), f32),
        scratch_types=[
            pltpu.VMEM((ETH,), i32),
            pltpu.VMEM((NCHH, CHUNK), i32),
            pltpu.VMEM((ETH,), f32),
            pltpu.VMEM((CHUNK, F), f32),
            pltpu.VMEM_SHARED((NPAD, F), f32),
            pltpu.SemaphoreType.DMA,
        ],
    )
    return kern(x_pad, src_h, dst2d_h, s_h)


_PREC = jax.lax.Precision.HIGHEST


def _dot(a, b):
    return jax.lax.dot(a, b, precision=_PREC, preferred_element_type=f32)


def _t1_body(deg_ref, cnt_ref, wg_ref, wl1_ref, lw_ref, bg_ref, wr1_ref,
             bl1_ref, lb_ref,
             dis_ref, invc_ref, wgv_ref, bgv_ref, wrv_ref, blv_ref):
    deg = deg_ref[0] + deg_ref[1] + 1.0
    dis_ref[...] = jax.lax.rsqrt(deg)
    cnt = cnt_ref[0] + cnt_ref[1]
    invc_ref[...] = 1.0 / jnp.maximum(cnt, 1.0)
    v = _dot(wl1_ref[...], lw_ref[...])
    wgv_ref[...] = _dot(wg_ref[...], v)
    bgv_ref[...] = _dot(bg_ref[...], v)
    wrv_ref[...] = _dot(wr1_ref[...], lw_ref[...])
    blv_ref[...] = _dot(bl1_ref[...], lw_ref[...]) + lb_ref[...]


@jax.jit
def _t1(deg2, cnt2, Wg0, Wl1, lin_W, bg0, Wr1, bl1, lin_b):
    return pl.pallas_call(
        _t1_body,
        out_shape=(jax.ShapeDtypeStruct((NPAD,), f32),
                   jax.ShapeDtypeStruct((NPAD,), f32),
                   jax.ShapeDtypeStruct((F, O), f32),
                   jax.ShapeDtypeStruct((1, O), f32),
                   jax.ShapeDtypeStruct((H, O), f32),
                   jax.ShapeDtypeStruct((1, O), f32)),
    )(deg2, cnt2, Wg0, Wl1, lin_W, bg0.reshape(1, H), Wr1,
      bl1.reshape(1, H), lin_b.reshape(1, O))


RB = 1024
GRID = NPAD // RB


def _t2_body(s0_ref, invc_ref, xs_ref, wl_ref, bl_ref, wr_ref, o_ref):
    mean = s0_ref[...] * invc_ref[...]
    o_ref[...] = _dot(mean, wl_ref[...]) + bl_ref[...] + _dot(xs_ref[...], wr_ref[...])


@jax.jit
def _t2(s0_p, invc, xs_pad, Wl0, bl0, Wr0):
    return pl.pallas_call(
        _t2_body,
        grid=(GRID,),
        in_specs=[
            pl.BlockSpec((RB, F), lambda i: (i, 0)),
            pl.BlockSpec((RB, 1), lambda i: (i, 0)),
            pl.BlockSpec((RB, F), lambda i: (i, 0)),
            pl.BlockSpec((F, H), lambda i: (0, 0)),
            pl.BlockSpec((1, H), lambda i: (0, 0)),
            pl.BlockSpec((F, H), lambda i: (0, 0)),
        ],
        out_specs=pl.BlockSpec((RB, H), lambda i: (i, 0)),
        out_shape=jax.ShapeDtypeStruct((NPAD, H), f32),
    )(s0_p, invc, xs_pad, Wl0, bl0.reshape(1, H), Wr0)


def _t3_body(ra_ref, rb_ref, dis_ref, xr_ref, wgv_ref, bgv_ref, o_ref):
    dis = dis_ref[...]
    agg = ra_ref[0] + ra_ref[1] + rb_ref[0] + rb_ref[1]
    t = dis * agg + (dis * dis) * xr_ref[...]
    y = _dot(t, wgv_ref[...]) + bgv_ref[...]
    o_ref[...] = jnp.concatenate([y, jnp.zeros((RB, F - O), f32)], axis=1)


@jax.jit
def _t3(r0a, r0b, dis, xr_pad, WgV, bgV):
    return pl.pallas_call(
        _t3_body,
        grid=(GRID,),
        in_specs=[
            pl.BlockSpec((NC, RB, F), lambda i: (0, i, 0)),
            pl.BlockSpec((NC, RB, F), lambda i: (0, i, 0)),
            pl.BlockSpec((RB, 1), lambda i: (i, 0)),
            pl.BlockSpec((RB, F), lambda i: (i, 0)),
            pl.BlockSpec((F, O), lambda i: (0, 0)),
            pl.BlockSpec((1, O), lambda i: (0, 0)),
        ],
        out_specs=pl.BlockSpec((RB, F), lambda i: (i, 0)),
        out_shape=jax.ShapeDtypeStruct((NPAD, F), f32),
    )(r0a, r0b, dis, xr_pad, WgV, bgV)


def _t4_body(s1_ref, invc_ref, hs1_ref, wrv_ref, blv_ref, o_ref):
    aggy = s1_ref[:, :O]
    o_ref[...] = aggy * invc_ref[...] + _dot(hs1_ref[...], wrv_ref[...]) + blv_ref[...]


@jax.jit
def _t4(s1_p, invc, h_s1, WrV, blV):
    return pl.pallas_call(
        _t4_body,
        grid=(GRID,),
        in_specs=[
            pl.BlockSpec((RB, F), lambda i: (i, 0)),
            pl.BlockSpec((RB, 1), lambda i: (i, 0)),
            pl.BlockSpec((RB, H), lambda i: (i, 0)),
            pl.BlockSpec((H, O), lambda i: (0, 0)),
            pl.BlockSpec((1, O), lambda i: (0, 0)),
        ],
        out_specs=pl.BlockSpec((RB, O), lambda i: (i, 0)),
        out_shape=jax.ShapeDtypeStruct((NPAD, O), f32),
    )(s1_p, invc, h_s1, WrV, blV)


def kernel(x_region, x_subject, edge_index_rs, edge_index_rr, edge_weight_rr,
           sage_Wl_0, sage_bl_0, sage_Wr_0, gcn_W_0, gcn_b_0,
           sage_Wl_1, sage_bl_1, sage_Wr_1, gcn_W_1, gcn_b_1,
           lin_W, lin_b):
    pad_e = EPAD - E
    xr_pad = jnp.zeros((NPAD, F), f32).at[:N].set(x_region)
    xs_pad = jnp.zeros((NPAD, F), f32).at[:N].set(x_subject)

    pad_dst = N + (jnp.arange(pad_e, dtype=i32) % (NPAD - N))
    src_rs = jnp.concatenate([edge_index_rs[0], jnp.zeros((pad_e,), i32)])
    dst_rs = jnp.concatenate([edge_index_rs[1], pad_dst])
    src_rr = jnp.concatenate([edge_index_rr[0], jnp.zeros((pad_e,), i32)])
    dst_rr = jnp.concatenate([edge_index_rr[1], pad_dst])
    w_pad = jnp.concatenate([edge_weight_rr, jnp.zeros((pad_e,), f32)])

    def _ilv(a):
        return a.reshape(NCH, NC * NS, CHUNK).swapaxes(0, 1).reshape(EPAD)

    src_rs, dst_rs = _ilv(src_rs), _ilv(dst_rs)
    src_rr, dst_rr, w_pad = _ilv(src_rr), _ilv(dst_rr), _ilv(w_pad)

    dst_rs2d = dst_rs.reshape(NC * NS, NCH, CHUNK)
    dst_rr2d = dst_rr.reshape(NC * NS, NCH, CHUNK)

    deg2, cnt2 = _k0(dst_rs2d, dst_rr2d, w_pad)
    s0_p = _k1(xr_pad, src_rs, dst_rs2d, deg2)
    dis_flat, invc_flat, WgV, bgV, WrV, blV = _t1(
        deg2, cnt2, gcn_W_0, sage_Wl_1, lin_W, gcn_b_0, sage_Wr_1,
        sage_bl_1, lin_b)
    dis = dis_flat.reshape(NPAD, 1)
    invc = invc_flat.reshape(NPAD, 1)

    s_edge = _k1b(src_rr, w_pad, dis_flat, s0_p[:, :1, :1])
    r0_a = _k2(xr_pad, src_rr[:EPH], dst_rr2d[:NC * NS // 2].reshape(NC * NS, NCHH, CHUNK),
               s_edge[:EPH])
    r0_b = _k2(xr_pad, src_rr[EPH:], dst_rr2d[NC * NS // 2:].reshape(NC * NS, NCHH, CHUNK),
               s_edge[EPH:])
    s0 = s0_p[0] + s0_p[1]
    h_s1 = _t2(s0, invc, xs_pad, sage_Wl_0, sage_bl_0, sage_Wr_0)
    y_pad = _t3(r0_a, r0_b, dis, xr_pad, WgV, bgV)
    s1_p = _k1(y_pad, src_rs, dst_rs2d, r0_b[:, :1, :1])
    s1 = s1_p[0] + s1_p[1]
    out = _t4(s1, invc, h_s1, WrV, blV)
    return out[:N]


# speedup vs baseline: 1.0288x; 1.0288x over previous
import dataclasses

import jax
import jax.numpy as jnp
from jax import lax
from jax.experimental import pallas as pl
from jax.experimental.pallas import tpu as pltpu
from jax.experimental.pallas import tpu_sc as plsc

N = 10000
NPAD = 10240
F = 128
H = 256
O = 64
E = 160000
EPAD = 163840
NC, NS, L = 2, 16, 16
CHUNK = 128
ET = EPAD // (NC * NS)
NCH = ET // CHUNK
RPT = NPAD // NS

_mesh = plsc.VectorSubcoreMesh(core_axis_name="c", subcore_axis_name="s")
_SC_CP = pltpu.CompilerParams()
if "needs_layout_passes" in pltpu.CompilerParams.__dataclass_fields__:
    _SC_CP = dataclasses.replace(_SC_CP, needs_layout_passes=False)
f32 = jnp.float32
i32 = jnp.int32


def _zero_rows_f(tb_v):
    k = tb_v.shape[1] // L
    zf = jnp.zeros((L,), f32)

    @pl.loop(0, 128)
    def _(r):
        for fblk in range(k):
            tb_v[r, pl.ds(fblk * L, L)] = zf


def _k0_body(dstrs_hbm, dstrr_hbm, w_hbm,
             deg_out, cnt_out,
             dst_v, dstr_v, w_v, ones_v, st1_v,
             acc_deg, acc_cnt):
    c = lax.axis_index("c")
    s = lax.axis_index("s")
    wid = c * NS + s
    zf = jnp.zeros((L,), f32)
    of = jnp.ones((L,), f32)
    base = s * RPT

    @pl.loop(0, RPT, step=L)
    def _(r):
        st1_v[pl.ds(r, L)] = zf

    @pl.loop(0, CHUNK, step=L)
    def _(r):
        ones_v[pl.ds(r, L)] = of

    pltpu.sync_copy(st1_v, acc_deg.at[pl.ds(base, RPT)])
    pltpu.sync_copy(st1_v, acc_cnt.at[pl.ds(base, RPT)])

    pltpu.sync_copy(dstrs_hbm.at[wid], dst_v)
    pltpu.sync_copy(dstrr_hbm.at[wid], dstr_v)
    pltpu.sync_copy(w_hbm.at[pl.ds(wid * ET, ET)], w_v)

    plsc.subcore_barrier()

    @pl.loop(0, NCH)
    def _(j):
        pltpu.sync_copy(w_v.at[pl.ds(j * CHUNK, CHUNK)],
                        acc_deg.at[dstr_v.at[j]], add=True)
        pltpu.sync_copy(ones_v, acc_cnt.at[dst_v.at[j]], add=True)

    plsc.subcore_barrier()

    pltpu.sync_copy(acc_deg.at[pl.ds(base, RPT)], st1_v)
    pltpu.sync_copy(st1_v, deg_out.at[c, pl.ds(base, RPT)])
    pltpu.sync_copy(acc_cnt.at[pl.ds(base, RPT)], st1_v)
    pltpu.sync_copy(st1_v, cnt_out.at[c, pl.ds(base, RPT)])


@jax.jit
def _k0(dst_rs2d, dst_rr2d, w_pad):
    kern = pl.kernel(
        _k0_body,
        mesh=_mesh,
        compiler_params=_SC_CP,
        out_type=(
            jax.ShapeDtypeStruct((NC, NPAD), f32),
            jax.ShapeDtypeStruct((NC, NPAD), f32),
        ),
        scratch_types=[
            pltpu.VMEM((NCH, CHUNK), i32),
            pltpu.VMEM((NCH, CHUNK), i32),
            pltpu.VMEM((ET,), f32),
            pltpu.VMEM((CHUNK,), f32),
            pltpu.VMEM((RPT,), f32),
            pltpu.VMEM_SHARED((NPAD,), f32),
            pltpu.VMEM_SHARED((NPAD,), f32),
        ],
    )
    return kern(dst_rs2d, dst_rr2d, w_pad)


def _k1_body(x_hbm, srcrs_hbm, dstrs_hbm, tok_hbm,
             s0_out,
             src_v, dst_v, r0, r1,
             acc_s0, ss):
    c = lax.axis_index("c")
    s = lax.axis_index("s")
    wid = c * NS + s
    base = s * RPT

    _zero_rows_f(r0)
    for r in range(RPT // 128):
        pltpu.sync_copy(r0, acc_s0.at[pl.ds(base + r * 128, 128)])

    pltpu.sync_copy(srcrs_hbm.at[pl.ds(wid * ET, ET)], src_v)
    pltpu.sync_copy(dstrs_hbm.at[wid], dst_v)

    plsc.subcore_barrier()

    rows = (r0, r1)

    @pl.loop(0, NCH // 2)
    def _(grp):
        j0 = grp * 2
        cps = [pltpu.async_copy(
            x_hbm.at[src_v.at[pl.ds((j0 + b) * CHUNK, CHUNK)]], rows[b], ss)
            for b in range(2)]
        for b in range(2):
            cps[b].wait()
        scps = [pltpu.async_copy(rows[b], acc_s0.at[dst_v.at[j0 + b]],
                                 ss, add=True) for b in range(2)]
        for b in range(2):
            scps[b].wait()

    plsc.subcore_barrier()

    for r in range(RPT // 128):
        row0 = base + r * 128
        pltpu.sync_copy(acc_s0.at[pl.ds(row0, 128)], r0)
        pltpu.sync_copy(r0, s0_out.at[c, pl.ds(row0, 128)])


@jax.jit
def _k1(x_pad, src_rs, dst_rs2d, tok):
    kern = pl.kernel(
        _k1_body,
        mesh=_mesh,
        compiler_params=_SC_CP,
        out_type=jax.ShapeDtypeStruct((NC, NPAD, F), f32),
        scratch_types=[
            pltpu.VMEM((ET,), i32),
            pltpu.VMEM((NCH, CHUNK), i32),
            pltpu.VMEM((CHUNK, F), f32),
            pltpu.VMEM((CHUNK, F), f32),
            pltpu.VMEM_SHARED((NPAD, F), f32),
            pltpu.SemaphoreType.DMA,
        ],
    )
    return kern(x_pad, src_rs, dst_rs2d, tok)


def _k1b_body(src_hbm, w_hbm, dis_hbm, tok_hbm,
              s_out,
              src_v, w_v, dis_v, st_v):
    c = lax.axis_index("c")
    s = lax.axis_index("s")
    wid = c * NS + s

    pltpu.sync_copy(src_hbm.at[pl.ds(wid * ET, ET)], src_v)
    pltpu.sync_copy(w_hbm.at[pl.ds(wid * ET, ET)], w_v)
    pltpu.sync_copy(dis_hbm, dis_v)

    @pl.loop(0, ET, step=L)
    def _(i):
        idx16 = src_v[pl.ds(i, L)]
        d16 = plsc.load_gather(dis_v, [idx16])
        st_v[pl.ds(i, L)] = d16 * w_v[pl.ds(i, L)]

    pltpu.sync_copy(st_v, s_out.at[pl.ds(wid * ET, ET)])


@jax.jit
def _k1b(src_rr, w_pad, dis_flat, tok):
    kern = pl.kernel(
        _k1b_body,
        mesh=_mesh,
        compiler_params=_SC_CP,
        out_type=jax.ShapeDtypeStruct((EPAD,), f32),
        scratch_types=[
            pltpu.VMEM((ET,), i32),
            pltpu.VMEM((ET,), f32),
            pltpu.VMEM((NPAD,), f32),
            pltpu.VMEM((ET,), f32),
        ],
    )
    return kern(src_rr, w_pad, dis_flat, tok)


EPH = EPAD // 2
ETH = EPH // (NC * NS)
NCHH = ETH // CHUNK


def _k2_body(x_hbm, src_hbm, dst_hbm, s_hbm,
             r0_out,
             src_v, dst_v, sch_v, r0,
             acc, ss):
    c = lax.axis_index("c")
    s = lax.axis_index("s")
    wid = c * NS + s
    zi = jnp.zeros((L,), i32)
    base = s * RPT

    _zero_rows_f(r0)
    for r in range(RPT // 128):
        pltpu.sync_copy(r0, acc.at[pl.ds(base + r * 128, 128)])

    pltpu.sync_copy(src_hbm.at[pl.ds(wid * ETH, ETH)], src_v)
    pltpu.sync_copy(dst_hbm.at[wid], dst_v)
    pltpu.sync_copy(s_hbm.at[pl.ds(wid * ETH, ETH)], sch_v)

    plsc.subcore_barrier()

    @pl.loop(0, NCHH)
    def _(j):
        pltpu.async_copy(x_hbm.at[src_v.at[pl.ds(j * CHUNK, CHUNK)]], r0, ss).wait()

        @pl.loop(0, CHUNK)
        def _(e):
            sp = plsc.load_gather(sch_v, [zi + (j * CHUNK + e)])
            for fblk in range(F // L):
                r0[e, pl.ds(fblk * L, L)] = r0[e, pl.ds(fblk * L, L)] * sp

        pltpu.sync_copy(r0, acc.at[dst_v.at[j]], add=True)

    plsc.subcore_barrier()

    for r in range(RPT // 128):
        row0 = base + r * 128
        pltpu.sync_copy(acc.at[pl.ds(row0, 128)], r0)
        pltpu.sync_copy(r0, r0_out.at[c, pl.ds(row0, 128)])


@jax.jit
def _k2(x_pad, src_h, dst2d_h, s_h):
    kern = pl.kernel(
        _k2_body,
        mesh=_mesh,
        compiler_params=_SC_CP,
        out_type=jax.ShapeDtypeStruct((NC, NPAD, F), f32),
        scratch_types=[
            pltpu.VMEM((ETH,), i32),
            pltpu.VMEM((NCHH, CHUNK), i32),
            pltpu.VMEM((ETH,), f32),
            pltpu.VMEM((CHUNK, F), f32),
            pltpu.VMEM_SHARED((NPAD, F), f32),
            pltpu.SemaphoreType.DMA,
        ],
    )
    return kern(x_pad, src_h, dst2d_h, s_h)


_PREC = jax.lax.Precision.HIGHEST


def _dot(a, b):
    return jax.lax.dot(a, b, precision=_PREC, preferred_element_type=f32)


def _t1_body(deg_ref, cnt_ref, wg_ref, wl1_ref, lw_ref, bg_ref, wr1_ref,
             bl1_ref, lb_ref,
             dis_ref, invc_ref, wgv_ref, bgv_ref, wrv_ref, blv_ref):
    deg = deg_ref[0] + deg_ref[1] + 1.0
    dis_ref[...] = jax.lax.rsqrt(deg)
    cnt = cnt_ref[0] + cnt_ref[1]
    invc_ref[...] = 1.0 / jnp.maximum(cnt, 1.0)
    v = _dot(wl1_ref[...], lw_ref[...])
    wgv_ref[...] = _dot(wg_ref[...], v)
    bgv_ref[...] = _dot(bg_ref[...], v)
    wrv_ref[...] = _dot(wr1_ref[...], lw_ref[...])
    blv_ref[...] = _dot(bl1_ref[...], lw_ref[...]) + lb_ref[...]


@jax.jit
def _t1(deg2, cnt2, Wg0, Wl1, lin_W, bg0, Wr1, bl1, lin_b):
    return pl.pallas_call(
        _t1_body,
        out_shape=(jax.ShapeDtypeStruct((NPAD,), f32),
                   jax.ShapeDtypeStruct((NPAD,), f32),
                   jax.ShapeDtypeStruct((F, O), f32),
                   jax.ShapeDtypeStruct((1, O), f32),
                   jax.ShapeDtypeStruct((H, O), f32),
                   jax.ShapeDtypeStruct((1, O), f32)),
    )(deg2, cnt2, Wg0, Wl1, lin_W, bg0.reshape(1, H), Wr1,
      bl1.reshape(1, H), lin_b.reshape(1, O))


RB = 1024
GRID = NPAD // RB


def _t2_body(s0_ref, invc_ref, xs_ref, wl_ref, bl_ref, wr_ref, o_ref):
    mean = s0_ref[...] * invc_ref[...]
    o_ref[...] = _dot(mean, wl_ref[...]) + bl_ref[...] + _dot(xs_ref[...], wr_ref[...])


@jax.jit
def _t2(s0_p, invc, xs_pad, Wl0, bl0, Wr0):
    return pl.pallas_call(
        _t2_body,
        grid=(GRID,),
        in_specs=[
            pl.BlockSpec((RB, F), lambda i: (i, 0)),
            pl.BlockSpec((RB, 1), lambda i: (i, 0)),
            pl.BlockSpec((RB, F), lambda i: (i, 0)),
            pl.BlockSpec((F, H), lambda i: (0, 0)),
            pl.BlockSpec((1, H), lambda i: (0, 0)),
            pl.BlockSpec((F, H), lambda i: (0, 0)),
        ],
        out_specs=pl.BlockSpec((RB, H), lambda i: (i, 0)),
        out_shape=jax.ShapeDtypeStruct((NPAD, H), f32),
    )(s0_p, invc, xs_pad, Wl0, bl0.reshape(1, H), Wr0)


def _t3_body(ra_ref, rb_ref, dis_ref, xr_ref, wgv_ref, bgv_ref, o_ref):
    dis = dis_ref[...]
    agg = ra_ref[0] + ra_ref[1] + rb_ref[0] + rb_ref[1]
    t = dis * agg + (dis * dis) * xr_ref[...]
    y = _dot(t, wgv_ref[...]) + bgv_ref[...]
    o_ref[...] = jnp.concatenate([y, jnp.zeros((RB, F - O), f32)], axis=1)


@jax.jit
def _t3(r0a, r0b, dis, xr_pad, WgV, bgV):
    return pl.pallas_call(
        _t3_body,
        grid=(GRID,),
        in_specs=[
            pl.BlockSpec((NC, RB, F), lambda i: (0, i, 0)),
            pl.BlockSpec((NC, RB, F), lambda i: (0, i, 0)),
            pl.BlockSpec((RB, 1), lambda i: (i, 0)),
            pl.BlockSpec((RB, F), lambda i: (i, 0)),
            pl.BlockSpec((F, O), lambda i: (0, 0)),
            pl.BlockSpec((1, O), lambda i: (0, 0)),
        ],
        out_specs=pl.BlockSpec((RB, F), lambda i: (i, 0)),
        out_shape=jax.ShapeDtypeStruct((NPAD, F), f32),
    )(r0a, r0b, dis, xr_pad, WgV, bgV)


def _t4_body(s1_ref, invc_ref, hs1_ref, wrv_ref, blv_ref, o_ref):
    aggy = s1_ref[:, :O]
    o_ref[...] = aggy * invc_ref[...] + _dot(hs1_ref[...], wrv_ref[...]) + blv_ref[...]


@jax.jit
def _t4(s1_p, invc, h_s1, WrV, blV):
    return pl.pallas_call(
        _t4_body,
        grid=(GRID,),
        in_specs=[
            pl.BlockSpec((RB, F), lambda i: (i, 0)),
            pl.BlockSpec((RB, 1), lambda i: (i, 0)),
            pl.BlockSpec((RB, H), lambda i: (i, 0)),
            pl.BlockSpec((H, O), lambda i: (0, 0)),
            pl.BlockSpec((1, O), lambda i: (0, 0)),
        ],
        out_specs=pl.BlockSpec((RB, O), lambda i: (i, 0)),
        out_shape=jax.ShapeDtypeStruct((NPAD, O), f32),
    )(s1_p, invc, h_s1, WrV, blV)


def kernel(x_region, x_subject, edge_index_rs, edge_index_rr, edge_weight_rr,
           sage_Wl_0, sage_bl_0, sage_Wr_0, gcn_W_0, gcn_b_0,
           sage_Wl_1, sage_bl_1, sage_Wr_1, gcn_W_1, gcn_b_1,
           lin_W, lin_b):
    pad_e = EPAD - E
    xr_pad = jnp.zeros((NPAD, F), f32).at[:N].set(x_region)
    xs_pad = jnp.zeros((NPAD, F), f32).at[:N].set(x_subject)

    pad_dst = N + (jnp.arange(pad_e, dtype=i32) % (NPAD - N))
    src_rs = jnp.concatenate([edge_index_rs[0], jnp.zeros((pad_e,), i32)])
    dst_rs = jnp.concatenate([edge_index_rs[1], pad_dst])
    src_rr = jnp.concatenate([edge_index_rr[0], jnp.zeros((pad_e,), i32)])
    dst_rr = jnp.concatenate([edge_index_rr[1], pad_dst])
    w_pad = jnp.concatenate([edge_weight_rr, jnp.zeros((pad_e,), f32)])

    def _ilv(a):
        return a.reshape(NCH, NC * NS, CHUNK).swapaxes(0, 1).reshape(EPAD)

    src_rs, dst_rs = _ilv(src_rs), _ilv(dst_rs)
    src_rr, dst_rr, w_pad = _ilv(src_rr), _ilv(dst_rr), _ilv(w_pad)

    dst_rs2d = dst_rs.reshape(NC * NS, NCH, CHUNK)
    dst_rr2d = dst_rr.reshape(NC * NS, NCH, CHUNK)

    deg2, cnt2 = _k0(dst_rs2d, dst_rr2d, w_pad)
    s0_p = _k1(xr_pad, src_rs, dst_rs2d, deg2)
    dis_flat, invc_flat, WgV, bgV, WrV, blV = _t1(
        deg2, cnt2, gcn_W_0, sage_Wl_1, lin_W, gcn_b_0, sage_Wr_1,
        sage_bl_1, lin_b)
    dis = dis_flat.reshape(NPAD, 1)
    invc = invc_flat.reshape(NPAD, 1)

    s_edge = _k1b(src_rr, w_pad, dis_flat, s0_p[:, :1, :1])
    r0_a = _k2(xr_pad, src_rr[:EPH], dst_rr2d[:NC * NS // 2].reshape(NC * NS, NCHH, CHUNK),
               s_edge[:EPH])
    r0_b = _k2(xr_pad, src_rr[EPH:], dst_rr2d[NC * NS // 2:].reshape(NC * NS, NCHH, CHUNK),
               s_edge[EPH:])
    s0 = s0_p[0] + s0_p[1]
    h_s1 = _t2(s0, invc, xs_pad, sage_Wl_0, sage_bl_0, sage_Wr_0)
    y_pad = _t3(r0_a, r0_b, dis, xr_pad, WgV, bgV)
    s1_p = _k1(y_pad, src_rs, dst_rs2d, r0_b[:, :1, :1])
    s1 = s1_p[0] + s1_p[1]
    out = _t4(s1, invc, h_s1, WrV, blV)
    return out[:N]


# speedup vs baseline: 1.0375x; 1.0084x over previous
import dataclasses

import jax
import jax.numpy as jnp
from jax import lax
from jax.experimental import pallas as pl
from jax.experimental.pallas import tpu as pltpu
from jax.experimental.pallas import tpu_sc as plsc

N = 10000
NPAD = 10240
F = 128
H = 256
O = 64
E = 160000
EPAD = 163840
NC, NS, L = 2, 16, 16
CHUNK = 128
ET = EPAD // (NC * NS)
NCH = ET // CHUNK
RPT = NPAD // NS

_mesh = plsc.VectorSubcoreMesh(core_axis_name="c", subcore_axis_name="s")
_SC_CP = pltpu.CompilerParams()
if "needs_layout_passes" in pltpu.CompilerParams.__dataclass_fields__:
    _SC_CP = dataclasses.replace(_SC_CP, needs_layout_passes=False)
f32 = jnp.float32
i32 = jnp.int32


def _zero_rows_f(tb_v):
    k = tb_v.shape[1] // L
    zf = jnp.zeros((L,), f32)

    @pl.loop(0, 128)
    def _(r):
        for fblk in range(k):
            tb_v[r, pl.ds(fblk * L, L)] = zf


def _k0_body(dstrs_hbm, dstrr_hbm, w_hbm,
             deg_out, cnt_out,
             dst_v, dstr_v, w_v, ones_v, st1_v,
             acc_deg, acc_cnt):
    c = lax.axis_index("c")
    s = lax.axis_index("s")
    wid = c * NS + s
    zf = jnp.zeros((L,), f32)
    of = jnp.ones((L,), f32)
    base = s * RPT

    @pl.loop(0, RPT, step=L)
    def _(r):
        st1_v[pl.ds(r, L)] = zf

    @pl.loop(0, CHUNK, step=L)
    def _(r):
        ones_v[pl.ds(r, L)] = of

    pltpu.sync_copy(st1_v, acc_deg.at[pl.ds(base, RPT)])
    pltpu.sync_copy(st1_v, acc_cnt.at[pl.ds(base, RPT)])

    pltpu.sync_copy(dstrs_hbm.at[wid], dst_v)
    pltpu.sync_copy(dstrr_hbm.at[wid], dstr_v)
    pltpu.sync_copy(w_hbm.at[pl.ds(wid * ET, ET)], w_v)

    plsc.subcore_barrier()

    @pl.loop(0, NCH)
    def _(j):
        pltpu.sync_copy(w_v.at[pl.ds(j * CHUNK, CHUNK)],
                        acc_deg.at[dstr_v.at[j]], add=True)
        pltpu.sync_copy(ones_v, acc_cnt.at[dst_v.at[j]], add=True)

    plsc.subcore_barrier()

    pltpu.sync_copy(acc_deg.at[pl.ds(base, RPT)], st1_v)
    pltpu.sync_copy(st1_v, deg_out.at[c, pl.ds(base, RPT)])
    pltpu.sync_copy(acc_cnt.at[pl.ds(base, RPT)], st1_v)
    pltpu.sync_copy(st1_v, cnt_out.at[c, pl.ds(base, RPT)])


@jax.jit
def _k0(dst_rs2d, dst_rr2d, w_pad):
    kern = pl.kernel(
        _k0_body,
        mesh=_mesh,
        compiler_params=_SC_CP,
        out_type=(
            jax.ShapeDtypeStruct((NC, NPAD), f32),
            jax.ShapeDtypeStruct((NC, NPAD), f32),
        ),
        scratch_types=[
            pltpu.VMEM((NCH, CHUNK), i32),
            pltpu.VMEM((NCH, CHUNK), i32),
            pltpu.VMEM((ET,), f32),
            pltpu.VMEM((CHUNK,), f32),
            pltpu.VMEM((RPT,), f32),
            pltpu.VMEM_SHARED((NPAD,), f32),
            pltpu.VMEM_SHARED((NPAD,), f32),
        ],
    )
    return kern(dst_rs2d, dst_rr2d, w_pad)


def _k1_body(x_hbm, srcrs_hbm, dstrs_hbm, tok_hbm,
             s0_out,
             src_v, dst_v, r0, r1,
             acc_s0, ss):
    c = lax.axis_index("c")
    s = lax.axis_index("s")
    wid = c * NS + s
    base = s * RPT

    _zero_rows_f(r0)
    for r in range(RPT // 128):
        pltpu.sync_copy(r0, acc_s0.at[pl.ds(base + r * 128, 128)])

    pltpu.sync_copy(srcrs_hbm.at[pl.ds(wid * ET, ET)], src_v)
    pltpu.sync_copy(dstrs_hbm.at[wid], dst_v)

    plsc.subcore_barrier()

    rows = (r0, r1)

    @pl.loop(0, NCH // 2)
    def _(grp):
        j0 = grp * 2
        cps = [pltpu.async_copy(
            x_hbm.at[src_v.at[pl.ds((j0 + b) * CHUNK, CHUNK)]], rows[b], ss)
            for b in range(2)]
        for b in range(2):
            cps[b].wait()
        scps = [pltpu.async_copy(rows[b], acc_s0.at[dst_v.at[j0 + b]],
                                 ss, add=True) for b in range(2)]
        for b in range(2):
            scps[b].wait()

    plsc.subcore_barrier()

    for r in range(RPT // 128):
        row0 = base + r * 128
        pltpu.sync_copy(acc_s0.at[pl.ds(row0, 128)], r0)
        pltpu.sync_copy(r0, s0_out.at[c, pl.ds(row0, 128)])


@jax.jit
def _k1(x_pad, src_rs, dst_rs2d, tok):
    kern = pl.kernel(
        _k1_body,
        mesh=_mesh,
        compiler_params=_SC_CP,
        out_type=jax.ShapeDtypeStruct((NC, NPAD, F), f32),
        scratch_types=[
            pltpu.VMEM((ET,), i32),
            pltpu.VMEM((NCH, CHUNK), i32),
            pltpu.VMEM((CHUNK, F), f32),
            pltpu.VMEM((CHUNK, F), f32),
            pltpu.VMEM_SHARED((NPAD, F), f32),
            pltpu.SemaphoreType.DMA,
        ],
    )
    return kern(x_pad, src_rs, dst_rs2d, tok)


def _k1b_body(src_hbm, w_hbm, dis_hbm, tok_hbm,
              s_out,
              src_v, w_v, dis_v, st_v):
    c = lax.axis_index("c")
    s = lax.axis_index("s")
    wid = c * NS + s

    pltpu.sync_copy(src_hbm.at[pl.ds(wid * ET, ET)], src_v)
    pltpu.sync_copy(w_hbm.at[pl.ds(wid * ET, ET)], w_v)
    pltpu.sync_copy(dis_hbm, dis_v)

    @pl.loop(0, ET, step=L)
    def _(i):
        idx16 = src_v[pl.ds(i, L)]
        d16 = plsc.load_gather(dis_v, [idx16])
        st_v[pl.ds(i, L)] = d16 * w_v[pl.ds(i, L)]

    pltpu.sync_copy(st_v, s_out.at[pl.ds(wid * ET, ET)])


@jax.jit
def _k1b(src_rr, w_pad, dis_flat, tok):
    kern = pl.kernel(
        _k1b_body,
        mesh=_mesh,
        compiler_params=_SC_CP,
        out_type=jax.ShapeDtypeStruct((EPAD,), f32),
        scratch_types=[
            pltpu.VMEM((ET,), i32),
            pltpu.VMEM((ET,), f32),
            pltpu.VMEM((NPAD,), f32),
            pltpu.VMEM((ET,), f32),
        ],
    )
    return kern(src_rr, w_pad, dis_flat, tok)


EPH = EPAD // 2
ETH = EPH // (NC * NS)
NCHH = ETH // CHUNK


def _k2_body(x_hbm, src_hbm, dst_hbm, s_hbm,
             r0_out,
             src_v, dst_v, sch_v, r0, r1,
             acc, ss):
    c = lax.axis_index("c")
    s = lax.axis_index("s")
    wid = c * NS + s
    zi = jnp.zeros((L,), i32)
    base = s * RPT

    _zero_rows_f(r0)
    for r in range(RPT // 128):
        pltpu.sync_copy(r0, acc.at[pl.ds(base + r * 128, 128)])

    pltpu.sync_copy(src_hbm.at[pl.ds(wid * ETH, ETH)], src_v)
    pltpu.sync_copy(dst_hbm.at[wid], dst_v)
    pltpu.sync_copy(s_hbm.at[pl.ds(wid * ETH, ETH)], sch_v)

    plsc.subcore_barrier()

    rows = (r0, r1)

    @pl.loop(0, NCHH // 2)
    def _(grp):
        j0 = grp * 2
        cps = [pltpu.async_copy(
            x_hbm.at[src_v.at[pl.ds((j0 + b) * CHUNK, CHUNK)]], rows[b], ss)
            for b in range(2)]
        for b in range(2):
            cps[b].wait()
        for b in range(2):
            rv = rows[b]

            @pl.loop(0, CHUNK)
            def _(e):
                sp = plsc.load_gather(sch_v, [zi + ((j0 + b) * CHUNK + e)])
                for fblk in range(F // L):
                    rv[e, pl.ds(fblk * L, L)] = rv[e, pl.ds(fblk * L, L)] * sp

        scps = [pltpu.async_copy(rows[b], acc.at[dst_v.at[j0 + b]],
                                 ss, add=True) for b in range(2)]
        for b in range(2):
            scps[b].wait()

    plsc.subcore_barrier()

    for r in range(RPT // 128):
        row0 = base + r * 128
        pltpu.sync_copy(acc.at[pl.ds(row0, 128)], r0)
        pltpu.sync_copy(r0, r0_out.at[c, pl.ds(row0, 128)])


@jax.jit
def _k2(x_pad, src_h, dst2d_h, s_h):
    kern = pl.kernel(
        _k2_body,
        mesh=_mesh,
        compiler_params=_SC_CP,
        out_type=jax.ShapeDtypeStruct((NC, NPAD, F), f32),
        scratch_types=[
            pltpu.VMEM((ETH,), i32),
            pltpu.VMEM((NCHH, CHUNK), i32),
            pltpu.VMEM((ETH,), f32),
            pltpu.VMEM((CHUNK, F), f32),
            pltpu.VMEM((CHUNK, F), f32),
            pltpu.VMEM_SHARED((NPAD, F), f32),
            pltpu.SemaphoreType.DMA,
        ],
    )
    return kern(x_pad, src_h, dst2d_h, s_h)


_PREC = jax.lax.Precision.HIGHEST


def _dot(a, b):
    return jax.lax.dot(a, b, precision=_PREC, preferred_element_type=f32)


def _t1_body(deg_ref, cnt_ref, wg_ref, wl1_ref, lw_ref, bg_ref, wr1_ref,
             bl1_ref, lb_ref,
             dis_ref, invc_ref, wgv_ref, bgv_ref, wrv_ref, blv_ref):
    deg = deg_ref[0] + deg_ref[1] + 1.0
    dis_ref[...] = jax.lax.rsqrt(deg)
    cnt = cnt_ref[0] + cnt_ref[1]
    invc_ref[...] = 1.0 / jnp.maximum(cnt, 1.0)
    v = _dot(wl1_ref[...], lw_ref[...])
    wgv_ref[...] = _dot(wg_ref[...], v)
    bgv_ref[...] = _dot(bg_ref[...], v)
    wrv_ref[...] = _dot(wr1_ref[...], lw_ref[...])
    blv_ref[...] = _dot(bl1_ref[...], lw_ref[...]) + lb_ref[...]


@jax.jit
def _t1(deg2, cnt2, Wg0, Wl1, lin_W, bg0, Wr1, bl1, lin_b):
    return pl.pallas_call(
        _t1_body,
        out_shape=(jax.ShapeDtypeStruct((NPAD,), f32),
                   jax.ShapeDtypeStruct((NPAD,), f32),
                   jax.ShapeDtypeStruct((F, O), f32),
                   jax.ShapeDtypeStruct((1, O), f32),
                   jax.ShapeDtypeStruct((H, O), f32),
                   jax.ShapeDtypeStruct((1, O), f32)),
    )(deg2, cnt2, Wg0, Wl1, lin_W, bg0.reshape(1, H), Wr1,
      bl1.reshape(1, H), lin_b.reshape(1, O))


RB = 1024
GRID = NPAD // RB


def _t2_body(s0_ref, invc_ref, xs_ref, wl_ref, bl_ref, wr_ref, o_ref):
    mean = s0_ref[...] * invc_ref[...]
    o_ref[...] = _dot(mean, wl_ref[...]) + bl_ref[...] + _dot(xs_ref[...], wr_ref[...])


@jax.jit
def _t2(s0_p, invc, xs_pad, Wl0, bl0, Wr0):
    return pl.pallas_call(
        _t2_body,
        grid=(GRID,),
        in_specs=[
            pl.BlockSpec((RB, F), lambda i: (i, 0)),
            pl.BlockSpec((RB, 1), lambda i: (i, 0)),
            pl.BlockSpec((RB, F), lambda i: (i, 0)),
            pl.BlockSpec((F, H), lambda i: (0, 0)),
            pl.BlockSpec((1, H), lambda i: (0, 0)),
            pl.BlockSpec((F, H), lambda i: (0, 0)),
        ],
        out_specs=pl.BlockSpec((RB, H), lambda i: (i, 0)),
        out_shape=jax.ShapeDtypeStruct((NPAD, H), f32),
    )(s0_p, invc, xs_pad, Wl0, bl0.reshape(1, H), Wr0)


def _t3_body(ra_ref, rb_ref, dis_ref, xr_ref, wgv_ref, bgv_ref, o_ref):
    dis = dis_ref[...]
    agg = ra_ref[0] + ra_ref[1] + rb_ref[0] + rb_ref[1]
    t = dis * agg + (dis * dis) * xr_ref[...]
    y = _dot(t, wgv_ref[...]) + bgv_ref[...]
    o_ref[...] = jnp.concatenate([y, jnp.zeros((RB, F - O), f32)], axis=1)


@jax.jit
def _t3(r0a, r0b, dis, xr_pad, WgV, bgV):
    return pl.pallas_call(
        _t3_body,
        grid=(GRID,),
        in_specs=[
            pl.BlockSpec((NC, RB, F), lambda i: (0, i, 0)),
            pl.BlockSpec((NC, RB, F), lambda i: (0, i, 0)),
            pl.BlockSpec((RB, 1), lambda i: (i, 0)),
            pl.BlockSpec((RB, F), lambda i: (i, 0)),
            pl.BlockSpec((F, O), lambda i: (0, 0)),
            pl.BlockSpec((1, O), lambda i: (0, 0)),
        ],
        out_specs=pl.BlockSpec((RB, F), lambda i: (i, 0)),
        out_shape=jax.ShapeDtypeStruct((NPAD, F), f32),
    )(r0a, r0b, dis, xr_pad, WgV, bgV)


def _t4_body(s1_ref, invc_ref, hs1_ref, wrv_ref, blv_ref, o_ref):
    aggy = s1_ref[:, :O]
    o_ref[...] = aggy * invc_ref[...] + _dot(hs1_ref[...], wrv_ref[...]) + blv_ref[...]


@jax.jit
def _t4(s1_p, invc, h_s1, WrV, blV):
    return pl.pallas_call(
        _t4_body,
        grid=(GRID,),
        in_specs=[
            pl.BlockSpec((RB, F), lambda i: (i, 0)),
            pl.BlockSpec((RB, 1), lambda i: (i, 0)),
            pl.BlockSpec((RB, H), lambda i: (i, 0)),
            pl.BlockSpec((H, O), lambda i: (0, 0)),
            pl.BlockSpec((1, O), lambda i: (0, 0)),
        ],
        out_specs=pl.BlockSpec((RB, O), lambda i: (i, 0)),
        out_shape=jax.ShapeDtypeStruct((NPAD, O), f32),
    )(s1_p, invc, h_s1, WrV, blV)


def kernel(x_region, x_subject, edge_index_rs, edge_index_rr, edge_weight_rr,
           sage_Wl_0, sage_bl_0, sage_Wr_0, gcn_W_0, gcn_b_0,
           sage_Wl_1, sage_bl_1, sage_Wr_1, gcn_W_1, gcn_b_1,
           lin_W, lin_b):
    pad_e = EPAD - E
    xr_pad = jnp.zeros((NPAD, F), f32).at[:N].set(x_region)
    xs_pad = jnp.zeros((NPAD, F), f32).at[:N].set(x_subject)

    pad_dst = N + (jnp.arange(pad_e, dtype=i32) % (NPAD - N))
    src_rs = jnp.concatenate([edge_index_rs[0], jnp.zeros((pad_e,), i32)])
    dst_rs = jnp.concatenate([edge_index_rs[1], pad_dst])
    src_rr = jnp.concatenate([edge_index_rr[0], jnp.zeros((pad_e,), i32)])
    dst_rr = jnp.concatenate([edge_index_rr[1], pad_dst])
    w_pad = jnp.concatenate([edge_weight_rr, jnp.zeros((pad_e,), f32)])

    def _ilv(a):
        return a.reshape(NCH, NC * NS, CHUNK).swapaxes(0, 1).reshape(EPAD)

    src_rs, dst_rs = _ilv(src_rs), _ilv(dst_rs)
    src_rr, dst_rr, w_pad = _ilv(src_rr), _ilv(dst_rr), _ilv(w_pad)

    dst_rs2d = dst_rs.reshape(NC * NS, NCH, CHUNK)
    dst_rr2d = dst_rr.reshape(NC * NS, NCH, CHUNK)

    deg2, cnt2 = _k0(dst_rs2d, dst_rr2d, w_pad)
    s0_p = _k1(xr_pad, src_rs, dst_rs2d, deg2)
    dis_flat, invc_flat, WgV, bgV, WrV, blV = _t1(
        deg2, cnt2, gcn_W_0, sage_Wl_1, lin_W, gcn_b_0, sage_Wr_1,
        sage_bl_1, lin_b)
    dis = dis_flat.reshape(NPAD, 1)
    invc = invc_flat.reshape(NPAD, 1)

    s_edge = _k1b(src_rr, w_pad, dis_flat, s0_p[:, :1, :1])
    r0_a = _k2(xr_pad, src_rr[:EPH], dst_rr2d[:NC * NS // 2].reshape(NC * NS, NCHH, CHUNK),
               s_edge[:EPH])
    r0_b = _k2(xr_pad, src_rr[EPH:], dst_rr2d[NC * NS // 2:].reshape(NC * NS, NCHH, CHUNK),
               s_edge[EPH:])
    s0 = s0_p[0] + s0_p[1]
    h_s1 = _t2(s0, invc, xs_pad, sage_Wl_0, sage_bl_0, sage_Wr_0)
    y_pad = _t3(r0_a, r0_b, dis, xr_pad, WgV, bgV)
    s1_p = _k1(y_pad, src_rs, dst_rs2d, r0_b[:, :1, :1])
    s1 = s1_p[0] + s1_p[1]
    out = _t4(s1, invc, h_s1, WrV, blV)
    return out[:N]


# speedup vs baseline: 1.0401x; 1.0025x over previous
import dataclasses

import jax
import jax.numpy as jnp
from jax import lax
from jax.experimental import pallas as pl
from jax.experimental.pallas import tpu as pltpu
from jax.experimental.pallas import tpu_sc as plsc

N = 10000
NPAD = 10240
F = 128
H = 256
O = 64
E = 160000
EPAD = 163840
NC, NS, L = 2, 16, 16
CHUNK = 128
ET = EPAD // (NC * NS)
NCH = ET // CHUNK
RPT = NPAD // NS

_mesh = plsc.VectorSubcoreMesh(core_axis_name="c", subcore_axis_name="s")
_SC_CP = pltpu.CompilerParams()
if "needs_layout_passes" in pltpu.CompilerParams.__dataclass_fields__:
    _SC_CP = dataclasses.replace(_SC_CP, needs_layout_passes=False)
f32 = jnp.float32
i32 = jnp.int32


def _zero_rows_f(tb_v):
    k = tb_v.shape[1] // L
    zf = jnp.zeros((L,), f32)

    @pl.loop(0, 128)
    def _(r):
        for fblk in range(k):
            tb_v[r, pl.ds(fblk * L, L)] = zf


def _k0_body(dstrs_hbm, dstrr_hbm, w_hbm,
             deg_out, cnt_out,
             dst_v, dstr_v, w_v, ones_v, st1_v,
             acc_deg, acc_cnt):
    c = lax.axis_index("c")
    s = lax.axis_index("s")
    wid = c * NS + s
    zf = jnp.zeros((L,), f32)
    of = jnp.ones((L,), f32)
    base = s * RPT

    @pl.loop(0, RPT, step=L)
    def _(r):
        st1_v[pl.ds(r, L)] = zf

    @pl.loop(0, CHUNK, step=L)
    def _(r):
        ones_v[pl.ds(r, L)] = of

    pltpu.sync_copy(st1_v, acc_deg.at[pl.ds(base, RPT)])
    pltpu.sync_copy(st1_v, acc_cnt.at[pl.ds(base, RPT)])

    pltpu.sync_copy(dstrs_hbm.at[wid], dst_v)
    pltpu.sync_copy(dstrr_hbm.at[wid], dstr_v)
    pltpu.sync_copy(w_hbm.at[pl.ds(wid * ET, ET)], w_v)

    plsc.subcore_barrier()

    @pl.loop(0, NCH)
    def _(j):
        pltpu.sync_copy(w_v.at[pl.ds(j * CHUNK, CHUNK)],
                        acc_deg.at[dstr_v.at[j]], add=True)
        pltpu.sync_copy(ones_v, acc_cnt.at[dst_v.at[j]], add=True)

    plsc.subcore_barrier()

    pltpu.sync_copy(acc_deg.at[pl.ds(base, RPT)], deg_out.at[c, pl.ds(base, RPT)])
    pltpu.sync_copy(acc_cnt.at[pl.ds(base, RPT)], cnt_out.at[c, pl.ds(base, RPT)])


@jax.jit
def _k0(dst_rs2d, dst_rr2d, w_pad):
    kern = pl.kernel(
        _k0_body,
        mesh=_mesh,
        compiler_params=_SC_CP,
        out_type=(
            jax.ShapeDtypeStruct((NC, NPAD), f32),
            jax.ShapeDtypeStruct((NC, NPAD), f32),
        ),
        scratch_types=[
            pltpu.VMEM((NCH, CHUNK), i32),
            pltpu.VMEM((NCH, CHUNK), i32),
            pltpu.VMEM((ET,), f32),
            pltpu.VMEM((CHUNK,), f32),
            pltpu.VMEM((RPT,), f32),
            pltpu.VMEM_SHARED((NPAD,), f32),
            pltpu.VMEM_SHARED((NPAD,), f32),
        ],
    )
    return kern(dst_rs2d, dst_rr2d, w_pad)


def _k1_body(x_hbm, srcrs_hbm, dstrs_hbm, tok_hbm,
             s0_out,
             src_v, dst_v, r0, r1,
             acc_s0, ss):
    c = lax.axis_index("c")
    s = lax.axis_index("s")
    wid = c * NS + s
    base = s * RPT

    _zero_rows_f(r0)
    for r in range(RPT // 128):
        pltpu.sync_copy(r0, acc_s0.at[pl.ds(base + r * 128, 128)])

    pltpu.sync_copy(srcrs_hbm.at[pl.ds(wid * ET, ET)], src_v)
    pltpu.sync_copy(dstrs_hbm.at[wid], dst_v)

    plsc.subcore_barrier()

    rows = (r0, r1)

    @pl.loop(0, NCH // 2)
    def _(grp):
        j0 = grp * 2
        cps = [pltpu.async_copy(
            x_hbm.at[src_v.at[pl.ds((j0 + b) * CHUNK, CHUNK)]], rows[b], ss)
            for b in range(2)]
        for b in range(2):
            cps[b].wait()
        scps = [pltpu.async_copy(rows[b], acc_s0.at[dst_v.at[j0 + b]],
                                 ss, add=True) for b in range(2)]
        for b in range(2):
            scps[b].wait()

    plsc.subcore_barrier()

    pltpu.sync_copy(acc_s0.at[pl.ds(base, RPT)], s0_out.at[c, pl.ds(base, RPT)])


@jax.jit
def _k1(x_pad, src_rs, dst_rs2d, tok):
    kern = pl.kernel(
        _k1_body,
        mesh=_mesh,
        compiler_params=_SC_CP,
        out_type=jax.ShapeDtypeStruct((NC, NPAD, F), f32),
        scratch_types=[
            pltpu.VMEM((ET,), i32),
            pltpu.VMEM((NCH, CHUNK), i32),
            pltpu.VMEM((CHUNK, F), f32),
            pltpu.VMEM((CHUNK, F), f32),
            pltpu.VMEM_SHARED((NPAD, F), f32),
            pltpu.SemaphoreType.DMA,
        ],
    )
    return kern(x_pad, src_rs, dst_rs2d, tok)


def _k1b_body(src_hbm, w_hbm, dis_hbm, tok_hbm,
              s_out,
              src_v, w_v, dis_v, st_v):
    c = lax.axis_index("c")
    s = lax.axis_index("s")
    wid = c * NS + s

    pltpu.sync_copy(src_hbm.at[pl.ds(wid * ET, ET)], src_v)
    pltpu.sync_copy(w_hbm.at[pl.ds(wid * ET, ET)], w_v)
    pltpu.sync_copy(dis_hbm, dis_v)

    @pl.loop(0, ET, step=L)
    def _(i):
        idx16 = src_v[pl.ds(i, L)]
        d16 = plsc.load_gather(dis_v, [idx16])
        st_v[pl.ds(i, L)] = d16 * w_v[pl.ds(i, L)]

    pltpu.sync_copy(st_v, s_out.at[pl.ds(wid * ET, ET)])


@jax.jit
def _k1b(src_rr, w_pad, dis_flat, tok):
    kern = pl.kernel(
        _k1b_body,
        mesh=_mesh,
        compiler_params=_SC_CP,
        out_type=jax.ShapeDtypeStruct((EPAD,), f32),
        scratch_types=[
            pltpu.VMEM((ET,), i32),
            pltpu.VMEM((ET,), f32),
            pltpu.VMEM((NPAD,), f32),
            pltpu.VMEM((ET,), f32),
        ],
    )
    return kern(src_rr, w_pad, dis_flat, tok)


EPH = EPAD // 2
ETH = EPH // (NC * NS)
NCHH = ETH // CHUNK


def _k2_body(x_hbm, src_hbm, dst_hbm, s_hbm,
             r0_out,
             src_v, dst_v, sch_v, r0, r1,
             acc, ss):
    c = lax.axis_index("c")
    s = lax.axis_index("s")
    wid = c * NS + s
    zi = jnp.zeros((L,), i32)
    base = s * RPT

    _zero_rows_f(r0)
    for r in range(RPT // 128):
        pltpu.sync_copy(r0, acc.at[pl.ds(base + r * 128, 128)])

    pltpu.sync_copy(src_hbm.at[pl.ds(wid * ETH, ETH)], src_v)
    pltpu.sync_copy(dst_hbm.at[wid], dst_v)
    pltpu.sync_copy(s_hbm.at[pl.ds(wid * ETH, ETH)], sch_v)

    plsc.subcore_barrier()

    rows = (r0, r1)

    @pl.loop(0, NCHH // 2)
    def _(grp):
        j0 = grp * 2
        cps = [pltpu.async_copy(
            x_hbm.at[src_v.at[pl.ds((j0 + b) * CHUNK, CHUNK)]], rows[b], ss)
            for b in range(2)]
        for b in range(2):
            cps[b].wait()
        for b in range(2):
            rv = rows[b]

            @pl.loop(0, CHUNK)
            def _(e):
                sp = plsc.load_gather(sch_v, [zi + ((j0 + b) * CHUNK + e)])
                for fblk in range(F // L):
                    rv[e, pl.ds(fblk * L, L)] = rv[e, pl.ds(fblk * L, L)] * sp

        scps = [pltpu.async_copy(rows[b], acc.at[dst_v.at[j0 + b]],
                                 ss, add=True) for b in range(2)]
        for b in range(2):
            scps[b].wait()

    plsc.subcore_barrier()

    pltpu.sync_copy(acc.at[pl.ds(base, RPT)], r0_out.at[c, pl.ds(base, RPT)])


@jax.jit
def _k2(x_pad, src_h, dst2d_h, s_h):
    kern = pl.kernel(
        _k2_body,
        mesh=_mesh,
        compiler_params=_SC_CP,
        out_type=jax.ShapeDtypeStruct((NC, NPAD, F), f32),
        scratch_types=[
            pltpu.VMEM((ETH,), i32),
            pltpu.VMEM((NCHH, CHUNK), i32),
            pltpu.VMEM((ETH,), f32),
            pltpu.VMEM((CHUNK, F), f32),
            pltpu.VMEM((CHUNK, F), f32),
            pltpu.VMEM_SHARED((NPAD, F), f32),
            pltpu.SemaphoreType.DMA,
        ],
    )
    return kern(x_pad, src_h, dst2d_h, s_h)


_PREC = jax.lax.Precision.HIGHEST


def _dot(a, b):
    return jax.lax.dot(a, b, precision=_PREC, preferred_element_type=f32)


def _t1_body(deg_ref, cnt_ref, wg_ref, wl1_ref, lw_ref, bg_ref, wr1_ref,
             bl1_ref, lb_ref,
             dis_ref, invc_ref, wgv_ref, bgv_ref, wrv_ref, blv_ref):
    deg = deg_ref[0] + deg_ref[1] + 1.0
    dis_ref[...] = jax.lax.rsqrt(deg)
    cnt = cnt_ref[0] + cnt_ref[1]
    invc_ref[...] = 1.0 / jnp.maximum(cnt, 1.0)
    v = _dot(wl1_ref[...], lw_ref[...])
    wgv_ref[...] = _dot(wg_ref[...], v)
    bgv_ref[...] = _dot(bg_ref[...], v)
    wrv_ref[...] = _dot(wr1_ref[...], lw_ref[...])
    blv_ref[...] = _dot(bl1_ref[...], lw_ref[...]) + lb_ref[...]


@jax.jit
def _t1(deg2, cnt2, Wg0, Wl1, lin_W, bg0, Wr1, bl1, lin_b):
    return pl.pallas_call(
        _t1_body,
        out_shape=(jax.ShapeDtypeStruct((NPAD,), f32),
                   jax.ShapeDtypeStruct((NPAD,), f32),
                   jax.ShapeDtypeStruct((F, O), f32),
                   jax.ShapeDtypeStruct((1, O), f32),
                   jax.ShapeDtypeStruct((H, O), f32),
                   jax.ShapeDtypeStruct((1, O), f32)),
    )(deg2, cnt2, Wg0, Wl1, lin_W, bg0.reshape(1, H), Wr1,
      bl1.reshape(1, H), lin_b.reshape(1, O))


RB = 1024
GRID = NPAD // RB


def _t2_body(s0_ref, invc_ref, xs_ref, wl_ref, bl_ref, wr_ref, o_ref):
    mean = s0_ref[...] * invc_ref[...]
    o_ref[...] = _dot(mean, wl_ref[...]) + bl_ref[...] + _dot(xs_ref[...], wr_ref[...])


@jax.jit
def _t2(s0_p, invc, xs_pad, Wl0, bl0, Wr0):
    return pl.pallas_call(
        _t2_body,
        grid=(GRID,),
        in_specs=[
            pl.BlockSpec((RB, F), lambda i: (i, 0)),
            pl.BlockSpec((RB, 1), lambda i: (i, 0)),
            pl.BlockSpec((RB, F), lambda i: (i, 0)),
            pl.BlockSpec((F, H), lambda i: (0, 0)),
            pl.BlockSpec((1, H), lambda i: (0, 0)),
            pl.BlockSpec((F, H), lambda i: (0, 0)),
        ],
        out_specs=pl.BlockSpec((RB, H), lambda i: (i, 0)),
        out_shape=jax.ShapeDtypeStruct((NPAD, H), f32),
    )(s0_p, invc, xs_pad, Wl0, bl0.reshape(1, H), Wr0)


def _t3_body(ra_ref, rb_ref, dis_ref, xr_ref, wgv_ref, bgv_ref, o_ref):
    dis = dis_ref[...]
    agg = ra_ref[0] + ra_ref[1] + rb_ref[0] + rb_ref[1]
    t = dis * agg + (dis * dis) * xr_ref[...]
    y = _dot(t, wgv_ref[...]) + bgv_ref[...]
    o_ref[...] = jnp.concatenate([y, jnp.zeros((RB, F - O), f32)], axis=1)


@jax.jit
def _t3(r0a, r0b, dis, xr_pad, WgV, bgV):
    return pl.pallas_call(
        _t3_body,
        grid=(GRID,),
        in_specs=[
            pl.BlockSpec((NC, RB, F), lambda i: (0, i, 0)),
            pl.BlockSpec((NC, RB, F), lambda i: (0, i, 0)),
            pl.BlockSpec((RB, 1), lambda i: (i, 0)),
            pl.BlockSpec((RB, F), lambda i: (i, 0)),
            pl.BlockSpec((F, O), lambda i: (0, 0)),
            pl.BlockSpec((1, O), lambda i: (0, 0)),
        ],
        out_specs=pl.BlockSpec((RB, F), lambda i: (i, 0)),
        out_shape=jax.ShapeDtypeStruct((NPAD, F), f32),
    )(r0a, r0b, dis, xr_pad, WgV, bgV)


def _t4_body(s1_ref, invc_ref, hs1_ref, wrv_ref, blv_ref, o_ref):
    aggy = s1_ref[:, :O]
    o_ref[...] = aggy * invc_ref[...] + _dot(hs1_ref[...], wrv_ref[...]) + blv_ref[...]


@jax.jit
def _t4(s1_p, invc, h_s1, WrV, blV):
    return pl.pallas_call(
        _t4_body,
        grid=(GRID,),
        in_specs=[
            pl.BlockSpec((RB, F), lambda i: (i, 0)),
            pl.BlockSpec((RB, 1), lambda i: (i, 0)),
            pl.BlockSpec((RB, H), lambda i: (i, 0)),
            pl.BlockSpec((H, O), lambda i: (0, 0)),
            pl.BlockSpec((1, O), lambda i: (0, 0)),
        ],
        out_specs=pl.BlockSpec((RB, O), lambda i: (i, 0)),
        out_shape=jax.ShapeDtypeStruct((NPAD, O), f32),
    )(s1_p, invc, h_s1, WrV, blV)


def kernel(x_region, x_subject, edge_index_rs, edge_index_rr, edge_weight_rr,
           sage_Wl_0, sage_bl_0, sage_Wr_0, gcn_W_0, gcn_b_0,
           sage_Wl_1, sage_bl_1, sage_Wr_1, gcn_W_1, gcn_b_1,
           lin_W, lin_b):
    pad_e = EPAD - E
    xr_pad = jnp.zeros((NPAD, F), f32).at[:N].set(x_region)
    xs_pad = jnp.zeros((NPAD, F), f32).at[:N].set(x_subject)

    pad_dst = N + (jnp.arange(pad_e, dtype=i32) % (NPAD - N))
    src_rs = jnp.concatenate([edge_index_rs[0], jnp.zeros((pad_e,), i32)])
    dst_rs = jnp.concatenate([edge_index_rs[1], pad_dst])
    src_rr = jnp.concatenate([edge_index_rr[0], jnp.zeros((pad_e,), i32)])
    dst_rr = jnp.concatenate([edge_index_rr[1], pad_dst])
    w_pad = jnp.concatenate([edge_weight_rr, jnp.zeros((pad_e,), f32)])

    def _ilv(a):
        return a.reshape(NCH, NC * NS, CHUNK).swapaxes(0, 1).reshape(EPAD)

    src_rs, dst_rs = _ilv(src_rs), _ilv(dst_rs)
    src_rr, dst_rr, w_pad = _ilv(src_rr), _ilv(dst_rr), _ilv(w_pad)

    dst_rs2d = dst_rs.reshape(NC * NS, NCH, CHUNK)
    dst_rr2d = dst_rr.reshape(NC * NS, NCH, CHUNK)

    deg2, cnt2 = _k0(dst_rs2d, dst_rr2d, w_pad)
    s0_p = _k1(xr_pad, src_rs, dst_rs2d, deg2)
    dis_flat, invc_flat, WgV, bgV, WrV, blV = _t1(
        deg2, cnt2, gcn_W_0, sage_Wl_1, lin_W, gcn_b_0, sage_Wr_1,
        sage_bl_1, lin_b)
    dis = dis_flat.reshape(NPAD, 1)
    invc = invc_flat.reshape(NPAD, 1)

    s_edge = _k1b(src_rr, w_pad, dis_flat, s0_p[:, :1, :1])
    r0_a = _k2(xr_pad, src_rr[:EPH], dst_rr2d[:NC * NS // 2].reshape(NC * NS, NCHH, CHUNK),
               s_edge[:EPH])
    r0_b = _k2(xr_pad, src_rr[EPH:], dst_rr2d[NC * NS // 2:].reshape(NC * NS, NCHH, CHUNK),
               s_edge[EPH:])
    s0 = s0_p[0] + s0_p[1]
    h_s1 = _t2(s0, invc, xs_pad, sage_Wl_0, sage_bl_0, sage_Wr_0)
    y_pad = _t3(r0_a, r0_b, dis, xr_pad, WgV, bgV)
    s1_p = _k1(y_pad, src_rs, dst_rs2d, r0_b[:, :1, :1])
    s1 = s1_p[0] + s1_p[1]
    out = _t4(s1, invc, h_s1, WrV, blV)
    return out[:N]
